# baseline (speedup 1.0000x reference)
import functools

import jax
import jax.numpy as jnp
from jax import lax
from jax.experimental import pallas as pl
from jax.experimental.pallas import tpu as pltpu
from jax.experimental.pallas import tpu_sc as plsc

B = 1024
SEQ = 50
D = 128
OUT = 100000
LTAB = 8
KBITS = 6
NCODE = LTAB * (1 << KBITS)
CAND = 256
BLK = 2048
NBLK = (OUT + BLK - 1) // BLK
OUTP = NBLK * BLK
CHUNK = 128
NCHUNK = BLK // CHUNK

_F32 = jnp.float32
_HI = lax.Precision.HIGHEST


_NC, _NS = 2, 16
_NW = _NC * _NS
_BPW = B // _NW
_SEQP = 56
_IPW = _BPW * _SEQP


def _emb_body(emb_hbm, x_hbm, out_hbm, idx_v, rows_v, acc_v, sem0, sem1):
    wid = lax.axis_index("s") * _NC + lax.axis_index("c")
    pltpu.sync_copy(x_hbm.at[pl.ds(wid * _IPW, _IPW)], idx_v)
    sems = (sem0, sem1)
    handles = [None, None]
    handles[0] = pltpu.async_copy(
        emb_hbm.at[idx_v.at[pl.ds(0, SEQ)]], rows_v.at[0], sems[0])
    for i in range(_BPW):
        if i + 1 < _BPW:
            nxt = (i + 1) % 2
            handles[nxt] = pltpu.async_copy(
                emb_hbm.at[idx_v.at[pl.ds((i + 1) * _SEQP, SEQ)]],
                rows_v.at[nxt], sems[nxt])
        handles[i % 2].wait()
        buf = rows_v.at[i % 2]
        accs = tuple(buf[0, pl.ds(c * 16, 16)] for c in range(8))

        def _srow(s, a):
            return tuple(a[c] + buf[s, pl.ds(c * 16, 16)] for c in range(8))

        accs = lax.fori_loop(1, SEQ, _srow, accs)
        for c in range(8):
            acc_v[i, pl.ds(c * 16, 16)] = accs[c]
    pltpu.sync_copy(acc_v, out_hbm.at[pl.ds(wid * _BPW, _BPW)])


def _embed(emb_table, x_flat):
    return pl.kernel(
        _emb_body,
        out_type=jax.ShapeDtypeStruct((B, D), _F32),
        mesh=plsc.VectorSubcoreMesh(core_axis_name="c", subcore_axis_name="s"),
        scratch_types=[
            pltpu.VMEM((_IPW,), jnp.int32),
            pltpu.VMEM((2, SEQ, D), _F32),
            pltpu.VMEM((_BPW, D), _F32),
            pltpu.SemaphoreType.DMA,
            pltpu.SemaphoreType.DMA,
        ],
    )(emb_table, x_flat)


def _pack_matrix():
    li = lax.broadcasted_iota(jnp.int32, (LTAB, LTAB * KBITS), 0)
    mi = lax.broadcasted_iota(jnp.int32, (LTAB, LTAB * KBITS), 1)
    val = (1 << (mi % KBITS)).astype(_F32)
    return jnp.where(mi // KBITS == li, val, 0.0)


def _codes_onehot(scores):
    bits = (scores > 0).astype(_F32)
    codes = lax.dot_general(_pack_matrix(), bits, (((1,), (0,)), ((), ())),
                            preferred_element_type=_F32)
    codes = codes.astype(jnp.int32)
    n = scores.shape[1]
    parts = []
    for l in range(LTAB):
        iot = lax.broadcasted_iota(jnp.int32, (1 << KBITS, n), 0)
        parts.append((iot == codes[l:l + 1, :]).astype(jnp.bfloat16))
    return jnp.concatenate(parts, axis=0)


def _q_body(raw_ref, bias_ref, projm_ref, q_ref, ohq_ref):
    raw = raw_ref[:, :]
    nrm = jnp.sqrt(jnp.sum(raw * raw, axis=1, keepdims=True))
    q = raw / nrm + bias_ref[0:1, :]
    q = jnp.maximum(q, 0.0)
    q_ref[:, :] = q.astype(jnp.bfloat16)
    sq = lax.dot_general(projm_ref[:, :], q, (((1,), (1,)), ((), ())),
                         preferred_element_type=_F32, precision=_HI)
    ohq_ref[:, :] = _codes_onehot(sq)


def _query_stage(raw, bias8, projm):
    return pl.pallas_call(
        _q_body,
        out_shape=(
            jax.ShapeDtypeStruct((B, D), jnp.bfloat16),
            jax.ShapeDtypeStruct((NCODE, B), jnp.bfloat16),
        ),
    )(raw, bias8, projm)


def _a_body(w_ref, projm_ref, ohq_ref, cnt_ref, thr_ref, hist_ref):
    j = pl.program_id(0)
    sw = lax.dot_general(projm_ref[:, :], w_ref[:, :], (((1,), (1,)), ((), ())),
                         preferred_element_type=_F32, precision=_HI)
    ohw = _codes_onehot(sw)
    cnt = lax.dot_general(ohw, ohq_ref[:, :], (((0,), (0,)), ((), ())),
                          preferred_element_type=_F32)
    nglc = j * BLK + lax.broadcasted_iota(jnp.int32, (BLK, 1), 0)
    cnt = jnp.where(nglc < OUT, cnt, -1.0)
    cnt_ref[:, :] = cnt.astype(jnp.int8)
    cnt_bf = cnt.astype(jnp.bfloat16)
    nvalid = jnp.minimum(OUT - j * BLK, BLK).astype(_F32)
    ones_row = jnp.ones((1, BLK), jnp.bfloat16)
    hrows = [None] * 9
    for v in range(1, 9, 2):
        pk = (jnp.where(cnt_bf == jnp.bfloat16(v),
                        jnp.bfloat16(1), jnp.bfloat16(0))
              + jnp.where(cnt_bf == jnp.bfloat16(v + 1),
                          jnp.bfloat16(4096), jnp.bfloat16(0)))
        both = lax.dot_general(ones_row, pk, (((1,), (0,)), ((), ())),
                               preferred_element_type=_F32)
        hi = jnp.floor(both * (1.0 / 4096.0))
        hrows[v] = both - hi * 4096.0
        hrows[v + 1] = hi
    hsum = hrows[1]
    for v in range(2, 9):
        hsum = hsum + hrows[v]
    hrows[0] = nvalid - hsum
    hist_blk = jnp.concatenate(hrows + [jnp.zeros((7, B), _F32)], axis=0)

    @pl.when(j == 0)
    def _():
        hist_ref[:, :] = jnp.zeros((16, B), _F32)

    hist_ref[:, :] += hist_blk

    @pl.when(j == NBLK - 1)
    def _():
        tot = hist_ref[:, :]
        suf = [jnp.zeros((1, B), _F32)] * 10
        for v in range(8, -1, -1):
            suf[v] = suf[v + 1] + tot[v:v + 1, :]
        t = jnp.zeros((1, B), _F32)
        for v in range(1, 9):
            t = t + (suf[v] >= float(CAND)).astype(_F32)
        snext = jnp.zeros((1, B), _F32)
        for v in range(9):
            snext = snext + suf[v + 1] * (t == float(v)).astype(_F32)
        kt = float(CAND) - snext
        thr_ref[:, :] = jnp.concatenate([t, kt, jnp.zeros((6, B), _F32)], axis=0)


def _counts_stage(W, projm, ohq):
    return pl.pallas_call(
        _a_body,
        grid=(NBLK,),
        in_specs=[
            pl.BlockSpec((BLK, D), lambda j: (j, 0)),
            pl.BlockSpec((LTAB * KBITS, D), lambda j: (0, 0)),
            pl.BlockSpec((NCODE, B), lambda j: (0, 0)),
        ],
        out_specs=[
            pl.BlockSpec((BLK, B), lambda j: (j, 0)),
            pl.BlockSpec((8, B), lambda j: (0, 0)),
        ],
        out_shape=(
            jax.ShapeDtypeStruct((OUTP, B), jnp.int8),
            jax.ShapeDtypeStruct((8, B), _F32),
        ),
        scratch_shapes=[pltpu.VMEM((16, B), _F32)],
    )(W, projm, ohq)


def _b_body(w_ref, q_ref, cnt_ref, bo_ref, thr_ref, y_ref, out_ref, acc_ref):
    j = pl.program_id(0)

    @pl.when(j == 0)
    def _():
        acc_ref[:, :] = jnp.concatenate(
            [jnp.zeros((1, B), _F32), jnp.full((1, B), -1.0, _F32),
             jnp.zeros((6, B), _F32)], axis=0)

    logits = lax.dot_general(w_ref[:, :], q_ref[:, :],
                             (((1,), (1,)), ((), ())),
                             preferred_element_type=_F32)
    logits = logits + bo_ref[:, :]
    cnt = cnt_ref[:, :].astype(_F32)
    t = thr_ref[0:1, :]
    kt = thr_ref[1:2, :]
    gt = cnt > t
    eq = cnt == t
    cnt_bf = cnt_ref[:, :].astype(jnp.bfloat16)
    eq_bf = jnp.where(cnt_bf == t.astype(jnp.bfloat16),
                      jnp.bfloat16(1), jnp.bfloat16(0))
    e = jnp.exp(logits)
    i0 = lax.broadcasted_iota(jnp.int32, (CHUNK, CHUNK), 0).astype(jnp.bfloat16)
    i1 = lax.broadcasted_iota(jnp.int32, (CHUNK, CHUNK), 1).astype(jnp.bfloat16)
    tri = jnp.where(i0 > i1, jnp.bfloat16(1), jnp.bfloat16(0))
    base = acc_ref[4:5, :]
    sums = []
    for c in range(NCHUNK):
        lo = c * CHUNK
        eq_c = eq_bf[lo:lo + CHUNK, :]
        pos_c = base + lax.dot_general(tri, eq_c, (((1,), (0,)), ((), ())),
                                       preferred_element_type=_F32)
        sel_c = jnp.logical_or(
            gt[lo:lo + CHUNK, :],
            jnp.logical_and(eq[lo:lo + CHUNK, :], pos_c < kt))
        sums.append(jnp.sum(jnp.where(sel_c, e[lo:lo + CHUNK, :], 0.0),
                            axis=0, keepdims=True))
        base = pos_c[CHUNK - 1:CHUNK, :] + eq_bf[lo + CHUNK - 1:lo + CHUNK,
                                                 :].astype(_F32)
    acc_ref[4:5, :] = base
    acc_ref[0:1, :] += jnp.sum(jnp.concatenate(sums, axis=0), axis=0,
                               keepdims=True)
    iotac = lax.broadcasted_iota(jnp.int32, (BLK, 1), 0)
    key = cnt * 131072.0 + ((131071 - j * BLK).astype(_F32)
                            - iotac.astype(_F32))
    mk = jnp.max(key, axis=0, keepdims=True)
    lf = jnp.sum(jnp.where(key == mk, logits, 0.0), axis=0, keepdims=True)
    upd = mk > acc_ref[1:2, :]
    acc_ref[2:3, :] = jnp.where(upd, lf, acc_ref[2:3, :])
    acc_ref[1:2, :] = jnp.where(upd, mk, acc_ref[1:2, :])
    yeq = (iotac + j * BLK) == y_ref[0:1, :]
    acc_ref[3:4, :] += jnp.sum(jnp.where(yeq, logits, 0.0), axis=0,
                               keepdims=True)

    @pl.when(j == NBLK - 1)
    def _():
        se = acc_ref[0:1, :]
        bl = acc_ref[2:3, :]
        ly = acc_ref[3:4, :]
        tot = se - jnp.exp(bl) + jnp.exp(ly)
        loss = jnp.sum(jnp.log(tot) - ly) * (1.0 / B)
        out_ref[:, :] = jnp.full((8, 128), loss, _F32)


def _loss_stage(W, q, cnt, bo_col, thr, y2):
    return pl.pallas_call(
        _b_body,
        grid=(NBLK,),
        in_specs=[
            pl.BlockSpec((BLK, D), lambda j: (j, 0)),
            pl.BlockSpec((B, D), lambda j: (0, 0)),
            pl.BlockSpec((BLK, B), lambda j: (j, 0)),
            pl.BlockSpec((BLK, 1), lambda j: (j, 0)),
            pl.BlockSpec((8, B), lambda j: (0, 0)),
            pl.BlockSpec((8, B), lambda j: (0, 0)),
        ],
        out_specs=pl.BlockSpec((8, 128), lambda j: (0, 0)),
        out_shape=jax.ShapeDtypeStruct((8, 128), _F32),
        scratch_shapes=[pltpu.VMEM((8, B), _F32)],
    )(W, q, cnt, bo_col, thr, y2)


def kernel(x, y, freeze, emb_table, bias, W, b_out, proj):
    x_flat = jnp.pad(x.astype(jnp.int32), ((0, 0), (0, _SEQP - SEQ))).reshape(-1)
    raw = _embed(emb_table, x_flat)
    bias8 = jnp.broadcast_to(bias.reshape(1, D), (8, D))
    projm = proj.reshape(LTAB * KBITS, D)
    q, ohq = _query_stage(raw, bias8, projm)
    cnt, thr = _counts_stage(W, projm, ohq)
    bo_col = jnp.pad(b_out, (0, OUTP - OUT)).reshape(OUTP, 1)
    y2 = jnp.broadcast_to(y.reshape(1, B).astype(jnp.int32), (8, B))
    lossb = _loss_stage(W.astype(jnp.bfloat16), q, cnt, bo_col, thr, y2)
    return lossb[0, 0]


# speedup vs baseline: 1.0256x; 1.0256x over previous
import functools

import jax
import jax.numpy as jnp
from jax import lax
from jax.experimental import pallas as pl
from jax.experimental.pallas import tpu as pltpu
from jax.experimental.pallas import tpu_sc as plsc

B = 1024
SEQ = 50
D = 128
OUT = 100000
LTAB = 8
KBITS = 6
NCODE = LTAB * (1 << KBITS)
CAND = 256
BLK = 2048
NBLK = (OUT + BLK - 1) // BLK
OUTP = NBLK * BLK
CHUNK = 128
NCHUNK = BLK // CHUNK

_F32 = jnp.float32
_HI = lax.Precision.HIGHEST


_NC, _NS = 2, 16
_NW = _NC * _NS
_BPW = B // _NW
_SEQP = 56
_IPW = _BPW * _SEQP


def _emb_body(emb_hbm, x_hbm, out_hbm, idx_v, rows_v, acc_v, sem0, sem1):
    wid = lax.axis_index("s") * _NC + lax.axis_index("c")
    pltpu.sync_copy(x_hbm.at[pl.ds(wid * _IPW, _IPW)], idx_v)
    sems = (sem0, sem1)
    handles = [None, None]
    handles[0] = pltpu.async_copy(
        emb_hbm.at[idx_v.at[pl.ds(0, SEQ)]], rows_v.at[0], sems[0])
    for i in range(_BPW):
        if i + 1 < _BPW:
            nxt = (i + 1) % 2
            handles[nxt] = pltpu.async_copy(
                emb_hbm.at[idx_v.at[pl.ds((i + 1) * _SEQP, SEQ)]],
                rows_v.at[nxt], sems[nxt])
        handles[i % 2].wait()
        buf = rows_v.at[i % 2]
        accs = tuple(buf[0, pl.ds(c * 16, 16)] for c in range(8))

        def _srow(s, a):
            return tuple(a[c] + buf[s, pl.ds(c * 16, 16)] for c in range(8))

        accs = lax.fori_loop(1, SEQ, _srow, accs)
        for c in range(8):
            acc_v[i, pl.ds(c * 16, 16)] = accs[c]
    pltpu.sync_copy(acc_v, out_hbm.at[pl.ds(wid * _BPW, _BPW)])


def _embed(emb_table, x_flat):
    return pl.kernel(
        _emb_body,
        out_type=jax.ShapeDtypeStruct((B, D), _F32),
        mesh=plsc.VectorSubcoreMesh(core_axis_name="c", subcore_axis_name="s"),
        scratch_types=[
            pltpu.VMEM((_IPW,), jnp.int32),
            pltpu.VMEM((2, SEQ, D), _F32),
            pltpu.VMEM((_BPW, D), _F32),
            pltpu.SemaphoreType.DMA,
            pltpu.SemaphoreType.DMA,
        ],
    )(emb_table, x_flat)


def _pack_matrix():
    li = lax.broadcasted_iota(jnp.int32, (LTAB, LTAB * KBITS), 0)
    mi = lax.broadcasted_iota(jnp.int32, (LTAB, LTAB * KBITS), 1)
    val = (1 << (mi % KBITS)).astype(_F32)
    return jnp.where(mi // KBITS == li, val, 0.0)


def _codes_onehot(scores):
    bits = (scores > 0).astype(_F32)
    codes = lax.dot_general(_pack_matrix(), bits, (((1,), (0,)), ((), ())),
                            preferred_element_type=_F32)
    codes = codes.astype(jnp.int32)
    n = scores.shape[1]
    parts = []
    for l in range(LTAB):
        iot = lax.broadcasted_iota(jnp.int32, (1 << KBITS, n), 0)
        parts.append((iot == codes[l:l + 1, :]).astype(jnp.bfloat16))
    return jnp.concatenate(parts, axis=0)


def _q_body(raw_ref, bias_ref, projm_ref, q_ref, ohq_ref):
    raw = raw_ref[:, :]
    nrm = jnp.sqrt(jnp.sum(raw * raw, axis=1, keepdims=True))
    q = raw / nrm + bias_ref[0:1, :]
    q = jnp.maximum(q, 0.0)
    q_ref[:, :] = q.astype(jnp.bfloat16)
    sq = lax.dot_general(projm_ref[:, :], q, (((1,), (1,)), ((), ())),
                         preferred_element_type=_F32, precision=_HI)
    ohq_ref[:, :] = _codes_onehot(sq)


def _query_stage(raw, bias8, projm):
    return pl.pallas_call(
        _q_body,
        out_shape=(
            jax.ShapeDtypeStruct((B, D), jnp.bfloat16),
            jax.ShapeDtypeStruct((NCODE, B), jnp.bfloat16),
        ),
    )(raw, bias8, projm)


def _a_body(w_ref, projm_ref, ohq_ref, cnt_ref, thr_ref, hist_ref):
    j = pl.program_id(0)
    sw = lax.dot_general(projm_ref[:, :], w_ref[:, :], (((1,), (1,)), ((), ())),
                         preferred_element_type=_F32, precision=_HI)
    ohw = _codes_onehot(sw)
    cnt = lax.dot_general(ohw, ohq_ref[:, :], (((0,), (0,)), ((), ())),
                          preferred_element_type=_F32)
    nglc = j * BLK + lax.broadcasted_iota(jnp.int32, (BLK, 1), 0)
    cnt = jnp.where(nglc < OUT, cnt, -1.0)
    cnt_ref[:, :] = cnt.astype(jnp.int8)
    cnt_bf = cnt.astype(jnp.bfloat16)
    nvalid = jnp.minimum(OUT - j * BLK, BLK).astype(_F32)
    ones_row = jnp.ones((1, BLK), jnp.bfloat16)
    hrows = [None] * 9
    for v in range(1, 9, 2):
        pk = (jnp.where(cnt_bf == jnp.bfloat16(v),
                        jnp.bfloat16(1), jnp.bfloat16(0))
              + jnp.where(cnt_bf == jnp.bfloat16(v + 1),
                          jnp.bfloat16(4096), jnp.bfloat16(0)))
        both = lax.dot_general(ones_row, pk, (((1,), (0,)), ((), ())),
                               preferred_element_type=_F32)
        hi = jnp.floor(both * (1.0 / 4096.0))
        hrows[v] = both - hi * 4096.0
        hrows[v + 1] = hi
    hsum = hrows[1]
    for v in range(2, 9):
        hsum = hsum + hrows[v]
    hrows[0] = nvalid - hsum
    hist_blk = jnp.concatenate(hrows + [jnp.zeros((7, B), _F32)], axis=0)

    @pl.when(j == 0)
    def _():
        hist_ref[:, :] = jnp.zeros((16, B), _F32)

    hist_ref[:, :] += hist_blk

    @pl.when(j == NBLK - 1)
    def _():
        tot = hist_ref[:, :]
        suf = [jnp.zeros((1, B), _F32)] * 10
        for v in range(8, -1, -1):
            suf[v] = suf[v + 1] + tot[v:v + 1, :]
        t = jnp.zeros((1, B), _F32)
        for v in range(1, 9):
            t = t + (suf[v] >= float(CAND)).astype(_F32)
        snext = jnp.zeros((1, B), _F32)
        for v in range(9):
            snext = snext + suf[v + 1] * (t == float(v)).astype(_F32)
        kt = float(CAND) - snext
        thr_ref[:, :] = jnp.concatenate([t, kt, jnp.zeros((6, B), _F32)], axis=0)


def _counts_stage(W, projm, ohq):
    return pl.pallas_call(
        _a_body,
        grid=(NBLK,),
        in_specs=[
            pl.BlockSpec((BLK, D), lambda j: (j, 0)),
            pl.BlockSpec((LTAB * KBITS, D), lambda j: (0, 0)),
            pl.BlockSpec((NCODE, B), lambda j: (0, 0)),
        ],
        out_specs=[
            pl.BlockSpec((BLK, B), lambda j: (j, 0)),
            pl.BlockSpec((8, B), lambda j: (0, 0)),
        ],
        out_shape=(
            jax.ShapeDtypeStruct((OUTP, B), jnp.int8),
            jax.ShapeDtypeStruct((8, B), _F32),
        ),
        scratch_shapes=[pltpu.VMEM((16, B), _F32)],
    )(W, projm, ohq)


def _b_body(w_ref, q_ref, cnt_ref, bo_ref, thr_ref, y_ref, out_ref, acc_ref):
    j = pl.program_id(0)

    @pl.when(j == 0)
    def _():
        acc_ref[:, :] = jnp.concatenate(
            [jnp.zeros((1, B), _F32), jnp.full((1, B), -1.0, _F32),
             jnp.zeros((6, B), _F32)], axis=0)

    logits = lax.dot_general(w_ref[:, :].astype(jnp.bfloat16), q_ref[:, :],
                             (((1,), (1,)), ((), ())),
                             preferred_element_type=_F32)
    logits = logits + bo_ref[:, :]
    cnt = cnt_ref[:, :].astype(_F32)
    t = thr_ref[0:1, :]
    kt = thr_ref[1:2, :]
    gt = cnt > t
    eq = cnt == t
    cnt_bf = cnt_ref[:, :].astype(jnp.bfloat16)
    eq_bf = jnp.where(cnt_bf == t.astype(jnp.bfloat16),
                      jnp.bfloat16(1), jnp.bfloat16(0))
    e = jnp.exp(logits)
    i0 = lax.broadcasted_iota(jnp.int32, (CHUNK, CHUNK), 0).astype(jnp.bfloat16)
    i1 = lax.broadcasted_iota(jnp.int32, (CHUNK, CHUNK), 1).astype(jnp.bfloat16)
    tri = jnp.where(i0 > i1, jnp.bfloat16(1), jnp.bfloat16(0))
    base = acc_ref[4:5, :]
    sums = []
    for c in range(NCHUNK):
        lo = c * CHUNK
        eq_c = eq_bf[lo:lo + CHUNK, :]
        pos_c = base + lax.dot_general(tri, eq_c, (((1,), (0,)), ((), ())),
                                       preferred_element_type=_F32)
        sel_c = jnp.logical_or(
            gt[lo:lo + CHUNK, :],
            jnp.logical_and(eq[lo:lo + CHUNK, :], pos_c < kt))
        sums.append(jnp.sum(jnp.where(sel_c, e[lo:lo + CHUNK, :], 0.0),
                            axis=0, keepdims=True))
        base = pos_c[CHUNK - 1:CHUNK, :] + eq_bf[lo + CHUNK - 1:lo + CHUNK,
                                                 :].astype(_F32)
    acc_ref[4:5, :] = base
    acc_ref[0:1, :] += jnp.sum(jnp.concatenate(sums, axis=0), axis=0,
                               keepdims=True)
    iotac = lax.broadcasted_iota(jnp.int32, (BLK, 1), 0)
    key = cnt * 131072.0 + ((131071 - j * BLK).astype(_F32)
                            - iotac.astype(_F32))
    mk = jnp.max(key, axis=0, keepdims=True)
    lf = jnp.sum(jnp.where(key == mk, logits, 0.0), axis=0, keepdims=True)
    upd = mk > acc_ref[1:2, :]
    acc_ref[2:3, :] = jnp.where(upd, lf, acc_ref[2:3, :])
    acc_ref[1:2, :] = jnp.where(upd, mk, acc_ref[1:2, :])
    yeq = (iotac + j * BLK) == y_ref[0:1, :]
    acc_ref[3:4, :] += jnp.sum(jnp.where(yeq, logits, 0.0), axis=0,
                               keepdims=True)

    @pl.when(j == NBLK - 1)
    def _():
        se = acc_ref[0:1, :]
        bl = acc_ref[2:3, :]
        ly = acc_ref[3:4, :]
        tot = se - jnp.exp(bl) + jnp.exp(ly)
        loss = jnp.sum(jnp.log(tot) - ly) * (1.0 / B)
        out_ref[:, :] = jnp.full((8, 128), loss, _F32)


def _loss_stage(W, q, cnt, bo_col, thr, y2):
    return pl.pallas_call(
        _b_body,
        grid=(NBLK,),
        in_specs=[
            pl.BlockSpec((BLK, D), lambda j: (j, 0)),
            pl.BlockSpec((B, D), lambda j: (0, 0)),
            pl.BlockSpec((BLK, B), lambda j: (j, 0)),
            pl.BlockSpec((BLK, 1), lambda j: (j, 0)),
            pl.BlockSpec((8, B), lambda j: (0, 0)),
            pl.BlockSpec((8, B), lambda j: (0, 0)),
        ],
        out_specs=pl.BlockSpec((8, 128), lambda j: (0, 0)),
        out_shape=jax.ShapeDtypeStruct((8, 128), _F32),
        scratch_shapes=[pltpu.VMEM((8, B), _F32)],
    )(W, q, cnt, bo_col, thr, y2)


def kernel(x, y, freeze, emb_table, bias, W, b_out, proj):
    x_flat = jnp.pad(x.astype(jnp.int32), ((0, 0), (0, _SEQP - SEQ))).reshape(-1)
    raw = _embed(emb_table, x_flat)
    bias8 = jnp.broadcast_to(bias.reshape(1, D), (8, D))
    projm = proj.reshape(LTAB * KBITS, D)
    q, ohq = _query_stage(raw, bias8, projm)
    cnt, thr = _counts_stage(W, projm, ohq)
    bo_col = jnp.pad(b_out, (0, OUTP - OUT)).reshape(OUTP, 1)
    y2 = jnp.broadcast_to(y.reshape(1, B).astype(jnp.int32), (8, B))
    lossb = _loss_stage(W, q, cnt, bo_col, thr, y2)
    return lossb[0, 0]
